# Initial kernel scaffold; baseline (speedup 1.0000x reference)
import jax
import jax.numpy as jnp
from jax.experimental import pallas as pl


def kernel(drug_feats, cell_feats, params, edge_index_syn, edge_index_add, edge_index_ant, edge_index_intra):
    raise NotImplementedError("write your pallas kernel here")


# speedup vs baseline: 15.0186x; 15.0186x over previous
import dataclasses
import functools

import jax
import jax.numpy as jnp
import numpy as np
from jax import lax
from jax.experimental import pallas as pl
from jax.experimental.pallas import tpu as pltpu
from jax.experimental.pallas import tpu_sc as plsc

N_NODES = 50499
NPAD = 50688
NROWT = NPAD // 16
IN_TOT = 192
H = 4
D = 64
E = 400000
NREL = 4
PART = 32
NSLAB = NREL * 8
EB = 128
NB = E // EB
ZR = 288

f32 = jnp.float32
i32 = jnp.int32


def _sc_compiler_params():
    cp = pltpu.CompilerParams()
    fields = pltpu.CompilerParams.__dataclass_fields__
    if "needs_layout_passes" in fields:
        cp = dataclasses.replace(cp, needs_layout_passes=False)
    if "use_tc_tiling_on_sc" in fields:
        cp = dataclasses.replace(cp, use_tc_tiling_on_sc=False)
    return cp


def _tab_body(x_ref, w_ref, tl_ref, tr_ref):
    m = jnp.dot(x_ref[...], w_ref[...], preferred_element_type=f32)
    z = jnp.zeros((m.shape[0], 12), f32)
    for r in range(NREL):
        tl_ref[r] = jnp.concatenate([m[:, r * 8:r * 8 + 4], z], axis=1)
        tr_ref[r] = jnp.concatenate([m[:, r * 8 + 4:r * 8 + 8], z], axis=1)


def _tc_tab(xp, wtab):
    ba = 512
    return pl.pallas_call(
        _tab_body,
        grid=(NPAD // ba,),
        in_specs=[
            pl.BlockSpec((ba, IN_TOT), lambda i: (i, 0)),
            pl.BlockSpec((IN_TOT, 32), lambda i: (0, 0)),
        ],
        out_specs=[
            pl.BlockSpec((NREL, ba, 16), lambda i: (0, i, 0)),
            pl.BlockSpec((NREL, ba, 16), lambda i: (0, i, 0)),
        ],
        out_shape=[
            jax.ShapeDtypeStruct((NREL, NPAD, 16), f32),
            jax.ShapeDtypeStruct((NREL, NPAD, 16), f32),
        ],
    )(xp, wtab)


def _feat_body(x_ref, w_ref, o_ref):
    v = jnp.dot(x_ref[...], w_ref[...], preferred_element_type=f32)
    for s in range(NSLAB):
        o_ref[s] = v[:, s * PART:(s + 1) * PART]


def _tc_feat(xp, wcat):
    bb = 256
    return pl.pallas_call(
        _feat_body,
        grid=(NPAD // bb,),
        in_specs=[
            pl.BlockSpec((bb, IN_TOT), lambda i: (i, 0)),
            pl.BlockSpec((IN_TOT, NREL * 256), lambda i: (0, 0)),
        ],
        out_specs=pl.BlockSpec((NSLAB, bb, PART), lambda i: (0, i, 0)),
        out_shape=jax.ShapeDtypeStruct((NSLAB, NPAD, PART), f32),
    )(xp, wcat)


def _sck1_body(src_hbm, dst_hbm, tl_hbm, tr_hbm, ee_hbm, den_hbm,
               srcv, dstv, idxv, gl, gr, eev, ee4, zb, acc):
    c = lax.axis_index("c")
    s = lax.axis_index("s")
    w = c * 16 + s
    row0 = s * NROWT
    riota = lax.iota(i32, 16) // 4
    ciota = lax.iota(i32, 16) % 4

    @pl.loop(0, ZR)
    def _zb(i):
        zb[i] = jnp.zeros((16,), f32)

    @pl.loop(0, EB)
    def _ze(i):
        eev[i] = jnp.zeros((16,), f32)

    @pl.loop(0, NREL)
    def _rel(r):
        @pl.loop(0, NROWT // ZR)
        def _zero(k):
            pltpu.sync_copy(zb, acc.at[pl.ds(row0 + k * ZR, ZR)])

        plsc.subcore_barrier()

        @pl.loop(0, 98)
        def _batches(j):
            b = w + j * 32

            @pl.when(b < NB)
            def _():
                eoff = r * E + b * EB
                pltpu.sync_copy(src_hbm.at[pl.ds(eoff, EB)], srcv)
                pltpu.sync_copy(dst_hbm.at[pl.ds(eoff, EB)], dstv)
                radd = r * NPAD

                @pl.loop(0, EB // 16)
                def _adj1(t):
                    sl = pl.ds(t * 16, 16)
                    idxv[sl] = srcv[sl] + radd

                pltpu.sync_copy(tl_hbm.at[idxv], gl)

                @pl.loop(0, EB // 16)
                def _adj2(t):
                    sl = pl.ds(t * 16, 16)
                    idxv[sl] = dstv[sl] + radd

                pltpu.sync_copy(tr_hbm.at[idxv], gr)

                @pl.loop(0, EB // 4)
                def _grp(eg):
                    ridx = riota + eg * 4
                    va = plsc.load_gather(gl, [ridx, ciota])
                    vb = plsc.load_gather(gr, [ridx, ciota])
                    v = va + vb
                    e = jnp.where(v > 0.0, v, 0.2 * v)
                    ee = jnp.exp(e)
                    ee4[pl.ds(eg * 16, 16)] = ee
                    plsc.store_scatter(eev, [ridx, ciota], ee)

                pltpu.sync_copy(eev, acc.at[dstv], add=True)
                pltpu.sync_copy(ee4.at[pl.ds(0, EB * 4)],
                                ee_hbm.at[pl.ds(eoff * 4, EB * 4)])

        plsc.subcore_barrier()
        pltpu.sync_copy(acc.at[pl.ds(row0, NROWT)],
                        den_hbm.at[c, r, pl.ds(row0, NROWT)])


def _sc_pass1(src_all, dst_all, tabl, tabr):
    mesh = plsc.VectorSubcoreMesh(core_axis_name="c", subcore_axis_name="s")
    kf = functools.partial(
        pl.kernel,
        mesh=mesh,
        compiler_params=_sc_compiler_params(),
        out_type=[
            jax.ShapeDtypeStruct((NREL * E * 4,), f32),
            jax.ShapeDtypeStruct((2, NREL, NPAD, 16), f32),
        ],
        scratch_types=[
            pltpu.VMEM((EB,), i32),
            pltpu.VMEM((EB,), i32),
            pltpu.VMEM((EB,), i32),
            pltpu.VMEM((EB, 16), f32),
            pltpu.VMEM((EB, 16), f32),
            pltpu.VMEM((EB, 16), f32),
            pltpu.VMEM((EB * 4,), f32),
            pltpu.VMEM((ZR, 16), f32),
            pltpu.VMEM_SHARED((NPAD, 16), f32),
        ],
    )
    return kf(_sck1_body)(src_all, dst_all, tabl, tabr)


def _sck2_body(src_hbm, dst_hbm, ee_hbm, feat_hbm, num_hbm,
               srcv, dstv, idxv, eev, rows, zb, acc):
    c = lax.axis_index("c")
    s = lax.axis_index("s")
    row0 = s * NROWT

    @pl.loop(0, ZR)
    def _zb(i):
        zb[i, pl.ds(0, 16)] = jnp.zeros((16,), f32)
        zb[i, pl.ds(16, 16)] = jnp.zeros((16,), f32)

    @pl.loop(0, NSLAB // 2)
    def _round(i):
        slab = 2 * i + c
        rel = slab // 8
        h = (slab % 8) // 2
        sbase = slab * NPAD

        @pl.loop(0, NROWT // ZR)
        def _zero(k):
            pltpu.sync_copy(zb, acc.at[pl.ds(row0 + k * ZR, ZR)])

        plsc.subcore_barrier()

        @pl.loop(0, 196)
        def _batches(j):
            b = s + j * 16

            @pl.when(b < NB)
            def _():
                eoff = rel * E + b * EB
                pltpu.sync_copy(src_hbm.at[pl.ds(eoff, EB)], srcv)
                pltpu.sync_copy(dst_hbm.at[pl.ds(eoff, EB)], dstv)
                pltpu.sync_copy(ee_hbm.at[pl.ds(eoff * 4, EB * 4)], eev)

                @pl.loop(0, EB // 16)
                def _adj(t):
                    sl = pl.ds(t * 16, 16)
                    idxv[sl] = srcv[sl] + sbase

                pltpu.sync_copy(feat_hbm.at[idxv], rows)

                @pl.loop(0, EB)
                def _edge(e):
                    gidx = jnp.full((16,), e * 4 + h, i32)
                    g = plsc.load_gather(eev, [gidx])
                    rows[e, pl.ds(0, 16)] = rows[e, pl.ds(0, 16)] * g
                    rows[e, pl.ds(16, 16)] = rows[e, pl.ds(16, 16)] * g

                pltpu.sync_copy(rows, acc.at[dstv], add=True)

        plsc.subcore_barrier()
        pltpu.sync_copy(acc.at[pl.ds(row0, NROWT)],
                        num_hbm.at[pl.ds(sbase + row0, NROWT)])


def _sc_pass2(src_all, dst_all, ee_flat, featp):
    mesh = plsc.VectorSubcoreMesh(core_axis_name="c", subcore_axis_name="s")
    kf = functools.partial(
        pl.kernel,
        mesh=mesh,
        compiler_params=_sc_compiler_params(),
        out_type=jax.ShapeDtypeStruct((NSLAB * NPAD, PART), f32),
        scratch_types=[
            pltpu.VMEM((EB,), i32),
            pltpu.VMEM((EB,), i32),
            pltpu.VMEM((EB,), i32),
            pltpu.VMEM((EB * 4,), f32),
            pltpu.VMEM((EB, PART), f32),
            pltpu.VMEM((ZR, PART), f32),
            pltpu.VMEM_SHARED((NPAD, PART), f32),
        ],
    )
    return kf(_sck2_body)(src_all, dst_all, ee_flat, featp)


def _final_body(num_ref, den_ref, x_ref, wq_ref, bq_ref, wk_ref, bk_ref,
                wvo_ref, cout_ref, brs_ref, o_ref):
    dn = den_ref[0] + den_ref[1]
    parts = []
    for r in range(NREL):
        for h in range(H):
            nmr = jnp.concatenate(
                [num_ref[r * 8 + 2 * h], num_ref[r * 8 + 2 * h + 1]], axis=1)
            dv = dn[r, :, h][:, None]
            parts.append(nmr / (dv + 1e-16) + brs_ref[r * 4 + h][None, :])
    rstf = jnp.concatenate(parts, axis=0)
    kf = jnp.dot(rstf, wk_ref[...].T, preferred_element_type=f32) + bk_ref[...]
    vo = jnp.dot(rstf, wvo_ref[...].T, preferred_element_type=f32)
    cell = x_ref[:, 128:]
    q = jnp.dot(cell, wq_ref[...].T, preferred_element_type=f32) + bq_ref[...]
    bc = cell.shape[0]
    kf3 = kf.reshape(16, bc, D)
    vo3 = vo.reshape(16, bc, D)
    sc = jnp.sum(kf3 * q[None], axis=-1)
    m = jnp.max(sc, axis=0, keepdims=True)
    ww = jnp.exp(sc - m)
    a = ww / jnp.sum(ww, axis=0, keepdims=True)
    o = jnp.sum(a[:, :, None] * vo3, axis=0)
    o_ref[...] = o + cout_ref[...]


def _tc_final(num3, den, xp, wq, bq, wk, bk, wvo, cout, brs):
    bc = 256
    full = lambda shape: pl.BlockSpec(shape, lambda i: tuple(0 for _ in shape))
    return pl.pallas_call(
        _final_body,
        grid=(NPAD // bc,),
        in_specs=[
            pl.BlockSpec((NSLAB, bc, PART), lambda i: (0, i, 0)),
            pl.BlockSpec((2, NREL, bc, 16), lambda i: (0, 0, i, 0)),
            pl.BlockSpec((bc, IN_TOT), lambda i: (i, 0)),
            full((D, D)), full((1, D)), full((D, D)), full((1, D)),
            full((D, D)), full((1, D)), full((16, D)),
        ],
        out_specs=pl.BlockSpec((bc, D), lambda i: (i, 0)),
        out_shape=jax.ShapeDtypeStruct((NPAD, D), f32),
    )(num3, den, xp, wq, bq, wk, bk, wvo, cout, brs)


RELS = ("syn", "add", "ant", "intra")


def kernel(drug_feats, cell_feats, params, edge_index_syn, edge_index_add,
           edge_index_ant, edge_index_intra):
    p = params
    c_num, cf_dim = cell_feats.shape
    d_num = (drug_feats.shape[0] - c_num) // (c_num + 1)

    pad = jnp.broadcast_to(
        cell_feats.reshape(c_num, 1, cf_dim),
        (c_num, d_num, cf_dim)).reshape(c_num * d_num, cf_dim)
    pad3 = jnp.broadcast_to(
        jnp.mean(cell_feats, axis=0, keepdims=True), (d_num, cf_dim))
    cell_final = jnp.concatenate([pad, cell_feats, pad3], axis=0)
    x = jnp.concatenate([drug_feats, cell_final], axis=1)
    xp = jnp.pad(x, ((0, NPAD - N_NODES), (0, 0)))

    wtab_cols = []
    for r in RELS:
        W = p["W_" + r].reshape(H, D, IN_TOT)
        wtab_cols.append(jnp.einsum("hd,hdi->ih", p["al_" + r], W))
        wtab_cols.append(jnp.einsum("hd,hdi->ih", p["ar_" + r], W))
    wtab = jnp.concatenate(wtab_cols, axis=1)
    wcat = jnp.concatenate([p["W_" + r].T for r in RELS], axis=1)

    ipw, ipb = p["in_proj_W"], p["in_proj_b"]
    wq_f = (ipw[:D] @ p["wq_W"]) / np.sqrt(D)
    bq_f = ((ipw[:D] @ p["wq_b"] + ipb[:D]) / np.sqrt(D)).reshape(1, D)
    wk_f = ipw[D:2 * D] @ p["wk_W"]
    bk_f = (ipw[D:2 * D] @ p["wk_b"] + ipb[D:2 * D]).reshape(1, D)
    wv = ipw[2 * D:]
    wvo_f = p["out_proj_W"] @ wv @ p["wv_W"]
    c_out = (p["out_proj_W"] @ (wv @ p["wv_b"] + ipb[2 * D:])
             + p["out_proj_b"]).reshape(1, D)
    brs = jnp.concatenate(
        [p["b_" + r].reshape(H, D) for r in RELS], axis=0)

    src_all = jnp.concatenate([
        edge_index_syn[0], edge_index_add[0],
        edge_index_ant[0], edge_index_intra[0]])
    dst_all = jnp.concatenate([
        edge_index_syn[1], edge_index_add[1],
        edge_index_ant[1], edge_index_intra[1]])

    tabl4, tabr4 = _tc_tab(xp, wtab)
    tabl = tabl4.reshape(NREL * NPAD, 16)
    tabr = tabr4.reshape(NREL * NPAD, 16)
    featp = _tc_feat(xp, wcat)
    ee_flat, den = _sc_pass1(src_all, dst_all, tabl, tabr)
    numer = _sc_pass2(src_all, dst_all, ee_flat,
                      featp.reshape(NSLAB * NPAD, PART))
    outp = _tc_final(numer.reshape(NSLAB, NPAD, PART), den, xp,
                     wq_f, bq_f, wk_f, bk_f, wvo_f, c_out, brs)
    return outp[:N_NODES]


# speedup vs baseline: 30.1578x; 2.0080x over previous
import dataclasses
import functools

import jax
import jax.numpy as jnp
import numpy as np
from jax import lax
from jax.experimental import pallas as pl
from jax.experimental.pallas import tpu as pltpu
from jax.experimental.pallas import tpu_sc as plsc

N_NODES = 50499
NPAD = 50688
NROWT = NPAD // 16
IN_TOT = 192
H = 4
D = 64
E = 400000
NREL = 4
PART = 32
NSLAB = NREL * 8
EB = 128
EPAD = 401408
NBP = EPAD // EB
ZR = 288

f32 = jnp.float32
i32 = jnp.int32


def _sc_compiler_params():
    cp = pltpu.CompilerParams()
    fields = pltpu.CompilerParams.__dataclass_fields__
    if "needs_layout_passes" in fields:
        cp = dataclasses.replace(cp, needs_layout_passes=False)
    if "use_tc_tiling_on_sc" in fields:
        cp = dataclasses.replace(cp, use_tc_tiling_on_sc=False)
    return cp


def _tab_body(x_ref, w_ref, tl_ref, tr_ref):
    m = jnp.dot(x_ref[...], w_ref[...], preferred_element_type=f32)
    z = jnp.zeros((m.shape[0], 12), f32)
    for r in range(NREL):
        tl_ref[r] = jnp.concatenate([m[:, r * 8:r * 8 + 4], z], axis=1)
        tr_ref[r] = jnp.concatenate([m[:, r * 8 + 4:r * 8 + 8], z], axis=1)


def _tc_tab(xp, wtab):
    ba = 512
    return pl.pallas_call(
        _tab_body,
        grid=(NPAD // ba,),
        in_specs=[
            pl.BlockSpec((ba, IN_TOT), lambda i: (i, 0)),
            pl.BlockSpec((IN_TOT, 32), lambda i: (0, 0)),
        ],
        out_specs=[
            pl.BlockSpec((NREL, ba, 16), lambda i: (0, i, 0)),
            pl.BlockSpec((NREL, ba, 16), lambda i: (0, i, 0)),
        ],
        out_shape=[
            jax.ShapeDtypeStruct((NREL, NPAD, 16), f32),
            jax.ShapeDtypeStruct((NREL, NPAD, 16), f32),
        ],
    )(xp, wtab)


def _feat_body(x_ref, w_ref, o_ref):
    v = jnp.dot(x_ref[...], w_ref[...], preferred_element_type=f32)
    for s in range(NSLAB):
        o_ref[s] = v[:, s * PART:(s + 1) * PART]


def _tc_feat(xp, wcat):
    bb = 256
    return pl.pallas_call(
        _feat_body,
        grid=(NPAD // bb,),
        in_specs=[
            pl.BlockSpec((bb, IN_TOT), lambda i: (i, 0)),
            pl.BlockSpec((IN_TOT, NREL * 256), lambda i: (0, 0)),
        ],
        out_specs=pl.BlockSpec((NSLAB, bb, PART), lambda i: (0, i, 0)),
        out_shape=jax.ShapeDtypeStruct((NSLAB, NPAD, PART), f32),
    )(xp, wcat)


def _sck1_body(src_hbm, dst_hbm, tl_hbm, tr_hbm, ee_hbm, den_hbm,
               srcv, dstv, idxv, gl, gr, eev, ee4, zb, acc):
    c = lax.axis_index("c")
    s = lax.axis_index("s")
    w = c * 16 + s
    row0 = s * NROWT
    riota = lax.iota(i32, 16) // 4
    ciota = lax.iota(i32, 16) % 4

    @pl.loop(0, ZR)
    def _zb(i):
        zb[i] = jnp.zeros((16,), f32)

    @pl.loop(0, EB)
    def _ze(i):
        eev[i] = jnp.zeros((16,), f32)

    @pl.loop(0, NREL)
    def _rel(r):
        @pl.loop(0, NROWT // ZR)
        def _zero(k):
            pltpu.sync_copy(zb, acc.at[pl.ds(row0 + k * ZR, ZR)])

        plsc.subcore_barrier()

        @pl.loop(0, NBP // 32)
        def _batches(j):
            b = w + j * 32
            eoff = r * EPAD + b * EB
            pltpu.sync_copy(src_hbm.at[pl.ds(eoff, EB)], srcv)
            pltpu.sync_copy(dst_hbm.at[pl.ds(eoff, EB)], dstv)
            radd = r * NPAD

            @pl.loop(0, EB // 16)
            def _adj1(t):
                sl = pl.ds(t * 16, 16)
                idxv[sl] = srcv[sl] + radd

            pltpu.sync_copy(tl_hbm.at[idxv], gl)

            @pl.loop(0, EB // 16)
            def _adj2(t):
                sl = pl.ds(t * 16, 16)
                idxv[sl] = dstv[sl] + radd

            pltpu.sync_copy(tr_hbm.at[idxv], gr)

            @pl.loop(0, EB // 4)
            def _grp(eg):
                ridx = riota + eg * 4
                va = plsc.load_gather(gl, [ridx, ciota])
                vb = plsc.load_gather(gr, [ridx, ciota])
                v = va + vb
                e = jnp.where(v > 0.0, v, 0.2 * v)
                ee = jnp.exp(e)
                ee4[pl.ds(eg * 16, 16)] = ee
                plsc.store_scatter(eev, [ridx, ciota], ee)

            pltpu.sync_copy(eev, acc.at[dstv], add=True)
            pltpu.sync_copy(ee4.at[pl.ds(0, EB * 4)],
                            ee_hbm.at[pl.ds(eoff * 4, EB * 4)])

        plsc.subcore_barrier()
        pltpu.sync_copy(acc.at[pl.ds(row0, NROWT)],
                        den_hbm.at[c, r, pl.ds(row0, NROWT)])


def _sc_pass1(src_all, dst_all, tabl, tabr):
    mesh = plsc.VectorSubcoreMesh(core_axis_name="c", subcore_axis_name="s")
    kf = functools.partial(
        pl.kernel,
        mesh=mesh,
        compiler_params=_sc_compiler_params(),
        out_type=[
            jax.ShapeDtypeStruct((NREL * EPAD * 4,), f32),
            jax.ShapeDtypeStruct((2, NREL, NPAD, 16), f32),
        ],
        scratch_types=[
            pltpu.VMEM((EB,), i32),
            pltpu.VMEM((EB,), i32),
            pltpu.VMEM((EB,), i32),
            pltpu.VMEM((EB, 16), f32),
            pltpu.VMEM((EB, 16), f32),
            pltpu.VMEM((EB, 16), f32),
            pltpu.VMEM((EB * 4,), f32),
            pltpu.VMEM((ZR, 16), f32),
            pltpu.VMEM_SHARED((NPAD, 16), f32),
        ],
    )
    return kf(_sck1_body)(src_all, dst_all, tabl, tabr)


NJ = NBP // 16


def _sck2_body(src_hbm, dst_hbm, ee_hbm, feat_hbm, num_hbm,
               srcv, dstv, eev, idxv, dstS, rows, zb, acc, sA, sG, sS):
    c = lax.axis_index("c")
    s = lax.axis_index("s")
    row0 = s * NROWT

    @pl.loop(0, ZR)
    def _zb(i):
        zb[i, pl.ds(0, 16)] = jnp.zeros((16,), f32)
        zb[i, pl.ds(16, 16)] = jnp.zeros((16,), f32)

    @pl.loop(0, NSLAB // 2)
    def _round(i):
        slab = 2 * i + c
        rel = slab // 8
        h = (slab % 8) // 2
        sbase = slab * NPAD
        ebase = rel * EPAD

        @pl.loop(0, NROWT // ZR)
        def _zero(k):
            pltpu.sync_copy(zb, acc.at[pl.ds(row0 + k * ZR, ZR)])

        plsc.subcore_barrier()

        def issue_in(j, sl):
            eoff = ebase + (s + j * 16) * EB
            pltpu.async_copy(src_hbm.at[pl.ds(eoff, EB)], srcv.at[sl],
                             sA.at[sl])
            pltpu.async_copy(dst_hbm.at[pl.ds(eoff, EB)], dstv.at[sl],
                             sA.at[sl])
            pltpu.async_copy(ee_hbm.at[pl.ds(eoff * 4, EB * 4)], eev.at[sl],
                             sA.at[sl])

        def wait_in(sl):
            pltpu.make_async_copy(src_hbm.at[pl.ds(0, EB)], srcv.at[sl],
                                  sA.at[sl]).wait()
            pltpu.make_async_copy(dst_hbm.at[pl.ds(0, EB)], dstv.at[sl],
                                  sA.at[sl]).wait()
            pltpu.make_async_copy(ee_hbm.at[pl.ds(0, EB * 4)], eev.at[sl],
                                  sA.at[sl]).wait()

        def start_gather(j, gsl, isl):
            wait_in(isl)

            @pl.loop(0, EB // 16)
            def _adj(t):
                sl = pl.ds(t * 16, 16)
                idxv[gsl, sl] = srcv[isl, sl] + sbase

            @pl.when(j >= 2)
            def _():
                pltpu.make_async_copy(rows.at[gsl], acc.at[dstS.at[gsl]],
                                      sS.at[gsl]).wait()

            pltpu.async_copy(feat_hbm.at[idxv.at[gsl]], rows.at[gsl],
                             sG.at[gsl])

        def process(j, gsl, isl):
            pltpu.make_async_copy(feat_hbm.at[idxv.at[gsl]], rows.at[gsl],
                                  sG.at[gsl]).wait()

            @pl.loop(0, EB, unroll=8)
            def _edge(e):
                gidx = jnp.full((16,), e * 4 + h, i32)
                g = plsc.load_gather(eev.at[isl], [gidx])
                rows[gsl, e, pl.ds(0, 16)] = rows[gsl, e, pl.ds(0, 16)] * g
                rows[gsl, e, pl.ds(16, 16)] = rows[gsl, e, pl.ds(16, 16)] * g

            @pl.loop(0, EB // 16)
            def _cp(t):
                sl = pl.ds(t * 16, 16)
                dstS[gsl, sl] = dstv[isl, sl]

            pltpu.async_copy(rows.at[gsl], acc.at[dstS.at[gsl]], sS.at[gsl],
                             add=True)

            @pl.when(j + 4 < NJ)
            def _():
                issue_in(j + 4, isl)

        for m in range(4):
            issue_in(m, m)
        start_gather(0, 0, 0)

        @pl.loop(0, NJ // 4)
        def _batches(jj):
            for m in range(4):
                j = 4 * jj + m

                @pl.when(j + 1 < NJ)
                def _():
                    start_gather(j + 1, (m + 1) % 2, (m + 1) % 4)

                process(j, m % 2, m % 4)

        for gsl in range(2):
            pltpu.make_async_copy(rows.at[gsl], acc.at[dstS.at[gsl]],
                                  sS.at[gsl]).wait()

        plsc.subcore_barrier()
        pltpu.sync_copy(acc.at[pl.ds(row0, NROWT)],
                        num_hbm.at[pl.ds(sbase + row0, NROWT)])


def _sc_pass2(src_all, dst_all, ee_flat, featp):
    mesh = plsc.VectorSubcoreMesh(core_axis_name="c", subcore_axis_name="s")
    kf = functools.partial(
        pl.kernel,
        mesh=mesh,
        compiler_params=_sc_compiler_params(),
        out_type=jax.ShapeDtypeStruct((NSLAB * NPAD, PART), f32),
        scratch_types=[
            pltpu.VMEM((4, EB), i32),
            pltpu.VMEM((4, EB), i32),
            pltpu.VMEM((4, EB * 4), f32),
            pltpu.VMEM((2, EB), i32),
            pltpu.VMEM((2, EB), i32),
            pltpu.VMEM((2, EB, PART), f32),
            pltpu.VMEM((ZR, PART), f32),
            pltpu.VMEM_SHARED((NPAD, PART), f32),
            pltpu.SemaphoreType.DMA((4,)),
            pltpu.SemaphoreType.DMA((2,)),
            pltpu.SemaphoreType.DMA((2,)),
        ],
    )
    return kf(_sck2_body)(src_all, dst_all, ee_flat, featp)


def _final_body(num_ref, den_ref, x_ref, wq_ref, bq_ref, wk_ref, bk_ref,
                wvo_ref, cout_ref, brs_ref, o_ref):
    dn = den_ref[0] + den_ref[1]
    parts = []
    for r in range(NREL):
        for h in range(H):
            nmr = jnp.concatenate(
                [num_ref[r * 8 + 2 * h], num_ref[r * 8 + 2 * h + 1]], axis=1)
            dv = dn[r, :, h][:, None]
            parts.append(nmr / (dv + 1e-16) + brs_ref[r * 4 + h][None, :])
    rstf = jnp.concatenate(parts, axis=0)
    kf = jnp.dot(rstf, wk_ref[...].T, preferred_element_type=f32) + bk_ref[...]
    vo = jnp.dot(rstf, wvo_ref[...].T, preferred_element_type=f32)
    cell = x_ref[:, 128:]
    q = jnp.dot(cell, wq_ref[...].T, preferred_element_type=f32) + bq_ref[...]
    bc = cell.shape[0]
    kf3 = kf.reshape(16, bc, D)
    vo3 = vo.reshape(16, bc, D)
    sc = jnp.sum(kf3 * q[None], axis=-1)
    m = jnp.max(sc, axis=0, keepdims=True)
    ww = jnp.exp(sc - m)
    a = ww / jnp.sum(ww, axis=0, keepdims=True)
    o = jnp.sum(a[:, :, None] * vo3, axis=0)
    o_ref[...] = o + cout_ref[...]


def _tc_final(num3, den, xp, wq, bq, wk, bk, wvo, cout, brs):
    bc = 256
    full = lambda shape: pl.BlockSpec(shape, lambda i: tuple(0 for _ in shape))
    return pl.pallas_call(
        _final_body,
        grid=(NPAD // bc,),
        in_specs=[
            pl.BlockSpec((NSLAB, bc, PART), lambda i: (0, i, 0)),
            pl.BlockSpec((2, NREL, bc, 16), lambda i: (0, 0, i, 0)),
            pl.BlockSpec((bc, IN_TOT), lambda i: (i, 0)),
            full((D, D)), full((1, D)), full((D, D)), full((1, D)),
            full((D, D)), full((1, D)), full((16, D)),
        ],
        out_specs=pl.BlockSpec((bc, D), lambda i: (i, 0)),
        out_shape=jax.ShapeDtypeStruct((NPAD, D), f32),
    )(num3, den, xp, wq, bq, wk, bk, wvo, cout, brs)


RELS = ("syn", "add", "ant", "intra")


def kernel(drug_feats, cell_feats, params, edge_index_syn, edge_index_add,
           edge_index_ant, edge_index_intra):
    p = params
    c_num, cf_dim = cell_feats.shape
    d_num = (drug_feats.shape[0] - c_num) // (c_num + 1)

    pad = jnp.broadcast_to(
        cell_feats.reshape(c_num, 1, cf_dim),
        (c_num, d_num, cf_dim)).reshape(c_num * d_num, cf_dim)
    pad3 = jnp.broadcast_to(
        jnp.mean(cell_feats, axis=0, keepdims=True), (d_num, cf_dim))
    cell_final = jnp.concatenate([pad, cell_feats, pad3], axis=0)
    x = jnp.concatenate([drug_feats, cell_final], axis=1)
    xp = jnp.pad(x, ((0, NPAD - N_NODES), (0, 0)))

    wtab_cols = []
    for r in RELS:
        W = p["W_" + r].reshape(H, D, IN_TOT)
        wtab_cols.append(jnp.einsum("hd,hdi->ih", p["al_" + r], W))
        wtab_cols.append(jnp.einsum("hd,hdi->ih", p["ar_" + r], W))
    wtab = jnp.concatenate(wtab_cols, axis=1)
    wcat = jnp.concatenate([p["W_" + r].T for r in RELS], axis=1)

    ipw, ipb = p["in_proj_W"], p["in_proj_b"]
    wq_f = (ipw[:D] @ p["wq_W"]) / np.sqrt(D)
    bq_f = ((ipw[:D] @ p["wq_b"] + ipb[:D]) / np.sqrt(D)).reshape(1, D)
    wk_f = ipw[D:2 * D] @ p["wk_W"]
    bk_f = (ipw[D:2 * D] @ p["wk_b"] + ipb[D:2 * D]).reshape(1, D)
    wv = ipw[2 * D:]
    wvo_f = p["out_proj_W"] @ wv @ p["wv_W"]
    c_out = (p["out_proj_W"] @ (wv @ p["wv_b"] + ipb[2 * D:])
             + p["out_proj_b"]).reshape(1, D)
    brs = jnp.concatenate(
        [p["b_" + r].reshape(H, D) for r in RELS], axis=0)

    def _pad_edges(v):
        return jnp.pad(v, (0, EPAD - E), constant_values=N_NODES)

    src_all = jnp.concatenate([
        _pad_edges(edge_index_syn[0]), _pad_edges(edge_index_add[0]),
        _pad_edges(edge_index_ant[0]), _pad_edges(edge_index_intra[0])])
    dst_all = jnp.concatenate([
        _pad_edges(edge_index_syn[1]), _pad_edges(edge_index_add[1]),
        _pad_edges(edge_index_ant[1]), _pad_edges(edge_index_intra[1])])

    tabl4, tabr4 = _tc_tab(xp, wtab)
    tabl = tabl4.reshape(NREL * NPAD, 16)
    tabr = tabr4.reshape(NREL * NPAD, 16)
    featp = _tc_feat(xp, wcat)
    ee_flat, den = _sc_pass1(src_all, dst_all, tabl, tabr)
    numer = _sc_pass2(src_all, dst_all, ee_flat,
                      featp.reshape(NSLAB * NPAD, PART))
    outp = _tc_final(numer.reshape(NSLAB, NPAD, PART), den, xp,
                     wq_f, bq_f, wk_f, bk_f, wvo_f, c_out, brs)
    return outp[:N_NODES]
